# baseline (speedup 1.0000x reference)
import functools

import jax
import jax.numpy as jnp
from jax import lax
from jax.experimental import pallas as pl
from jax.experimental.pallas import tpu as pltpu
from jax.experimental.pallas import tpu_sc as plsc

_B = 16384
_J = 16
_K = 64
_NC = 2
_NS = 16
_NW = _NC * _NS
_BPW = _B // _NW
_EPW = _BPW * _J
_CH = 128
_NG = _EPW // _CH


def _sc_gather(xflat, idx0, idx2):
  mesh = plsc.VectorSubcoreMesh(core_axis_name="c", subcore_axis_name="s")

  @functools.partial(
      pl.kernel,
      mesh=mesh,
      out_type=jax.ShapeDtypeStruct((_B * _J,), jnp.float32),
      scratch_types=[
          pltpu.VMEM((_BPW,), jnp.int32),
          pltpu.VMEM((_BPW,), jnp.int32),
          pltpu.VMEM((_EPW,), jnp.int32),
          pltpu.VMEM((_EPW,), jnp.float32),
          pltpu.SemaphoreType.DMA,
      ],
  )
  def k(x_hbm, idx0_hbm, idx2_hbm, out_hbm, i0_v, i2_v, ib_v, o_v, sem):
    wid = lax.axis_index("s") * _NC + lax.axis_index("c")
    base = wid * _BPW
    pltpu.sync_copy(idx0_hbm.at[pl.ds(base, _BPW)], i0_v)
    pltpu.sync_copy(idx2_hbm.at[pl.ds(base, _BPW)], i2_v)

    lane = lax.iota(jnp.int32, 16)

    def build(c, carry):
      fb = i0_v[pl.ds(c * 16, 16)] * (_J * _K) + i2_v[pl.ds(c * 16, 16)]
      for r in range(16):
        ib_v[pl.ds(c * (16 * _J) + r * _J, _J)] = fb[r] + _K * lane
      return carry

    lax.fori_loop(0, _BPW // 16, build, 0)

    def fire(g, carry):
      pltpu.async_copy(
          x_hbm.at[ib_v.at[pl.ds(g * _CH, _CH)]],
          o_v.at[pl.ds(g * _CH, _CH)],
          sem,
      ).wait()
      return carry

    lax.fori_loop(0, _NG, fire, 0)

    pltpu.sync_copy(o_v, out_hbm.at[pl.ds(wid * _EPW, _EPW)])

  return k(xflat, idx0, idx2)


def kernel(x, idx0, idx2):
  xflat = jnp.reshape(x, (-1,))
  out = _sc_gather(xflat, idx0.astype(jnp.int32), idx2.astype(jnp.int32))
  return jnp.reshape(out, (_B, _J))


# speedup vs baseline: 1.0366x; 1.0366x over previous
import functools

import jax
import jax.numpy as jnp
from jax import lax
from jax.experimental import pallas as pl
from jax.experimental.pallas import tpu as pltpu
from jax.experimental.pallas import tpu_sc as plsc

_B = 16384
_J = 16
_K = 64
_NC = 2
_NS = 16
_NW = _NC * _NS
_BPW = _B // _NW
_EPW = _BPW * _J
_CH = 128
_NG = _EPW // _CH


def _sc_gather(xflat, idx0, idx2):
  mesh = plsc.VectorSubcoreMesh(core_axis_name="c", subcore_axis_name="s")

  @functools.partial(
      pl.kernel,
      mesh=mesh,
      out_type=jax.ShapeDtypeStruct((_B * _J,), jnp.float32),
      scratch_types=[
          pltpu.VMEM((_BPW,), jnp.int32),
          pltpu.VMEM((_BPW,), jnp.int32),
          pltpu.VMEM((_EPW,), jnp.int32),
          pltpu.VMEM((_EPW,), jnp.float32),
          pltpu.SemaphoreType.DMA,
      ],
  )
  def k(x_hbm, idx0_hbm, idx2_hbm, out_hbm, i0_v, i2_v, ib_v, o_v, sem):
    wid = lax.axis_index("s") * _NC + lax.axis_index("c")
    base = wid * _BPW
    pltpu.sync_copy(idx0_hbm.at[pl.ds(base, _BPW)], i0_v)
    pltpu.sync_copy(idx2_hbm.at[pl.ds(base, _BPW)], i2_v)

    lane = lax.iota(jnp.int32, 16)

    def build(c, carry):
      fb = i0_v[pl.ds(c * 16, 16)] * (_J * _K) + i2_v[pl.ds(c * 16, 16)]
      for r in range(16):
        ib_v[pl.ds(c * (16 * _J) + r * _J, _J)] = fb[r] + _K * lane
      return carry

    lax.fori_loop(0, _BPW // 16, build, 0)

    pltpu.async_copy(x_hbm.at[ib_v], o_v, sem).wait()

    pltpu.sync_copy(o_v, out_hbm.at[pl.ds(wid * _EPW, _EPW)])

  return k(xflat, idx0, idx2)


def kernel(x, idx0, idx2):
  xflat = jnp.reshape(x, (-1,))
  out = _sc_gather(xflat, idx0.astype(jnp.int32), idx2.astype(jnp.int32))
  return jnp.reshape(out, (_B, _J))
